# baseline (speedup 1.0000x reference)
import jax
import jax.numpy as jnp
from jax import lax
from jax.experimental import pallas as pl
from jax.experimental.pallas import tpu as pltpu
from jax.experimental.pallas import tpu_sc as plsc

_NC = 2
_NS = 16
_NW = _NC * _NS


def _tc_kernel(attn_ref, c_ref, idx_ref, pooled_ref):
    coarse = attn_ref[0, 0]
    for h in range(1, 8):
        coarse = coarse + attn_ref[0, h]
    coarse = coarse * 0.125

    idx_ref[0] = jnp.argmax(coarse, axis=1, keepdims=True)

    n = lax.broadcasted_iota(jnp.int32, (4096, 256), 0)
    l = lax.broadcasted_iota(jnp.int32, (4096, 256), 1)
    s = (((n // 256) * 16 + (n % 64) // 4) == l).astype(jnp.float32)
    pooled = lax.dot_general(
        c_ref[0], s, (((1,), (0,)), ((), ())),
        preferred_element_type=jnp.float32)
    pooled_ref[0] = pooled * 0.0625


def _sc_expand(pooled_hbm, idx_hbm, out_hbm, rows_v, idx_v, out_v, sem):
    wid = lax.axis_index("s") * _NC + lax.axis_index("c")
    b = wid // 8
    ch0 = (wid % 8) * 24
    pltpu.sync_copy(pooled_hbm.at[b, pl.ds(ch0 * 256, 24 * 256)], rows_v)
    pltpu.sync_copy(idx_hbm.at[pl.ds(b * 256, 256)], idx_v)

    lane = lax.iota(jnp.int32, 16)
    expand_perms = [4 * q + lane // 4 for q in range(4)]
    idx_chunks = [idx_v[pl.ds(c * 16, 16)] for c in range(16)]

    copies = []
    for r in range(24):
        for c in range(16):
            g = plsc.load_gather(rows_v, [r * 256 + idx_chunks[c]])
            for q in range(4):
                t = jnp.take(g, expand_perms[q])
                for j in range(4):
                    out_v[r, pl.ds(256 * c + 64 * j + 16 * q, 16)] = t
        if r % 8 == 7:
            copies.append(pltpu.async_copy(
                out_v.at[pl.ds(r - 7, 8), :],
                out_hbm.at[b, pl.ds(ch0 + r - 7, 8), :], sem))
    for cp in copies:
        cp.wait()


def kernel(A, B, C, D, attn):
    Bn, Cc, H, W = C.shape
    N = H * W
    c2 = C.reshape(Bn, Cc, N)

    idx, pooled = pl.pallas_call(
        _tc_kernel,
        grid=(Bn,),
        in_specs=[
            pl.BlockSpec((1, 8, 256, 256), lambda bb: (bb, 0, 0, 0)),
            pl.BlockSpec((1, Cc, N), lambda bb: (bb, 0, 0)),
        ],
        out_specs=[
            pl.BlockSpec((1, 256, 1), lambda bb: (bb, 0, 0)),
            pl.BlockSpec((1, Cc, 256), lambda bb: (bb, 0, 0)),
        ],
        out_shape=[
            jax.ShapeDtypeStruct((Bn, 256, 1), jnp.int32),
            jax.ShapeDtypeStruct((Bn, Cc, 256), jnp.float32),
        ],
    )(attn, c2)

    mesh = plsc.VectorSubcoreMesh(core_axis_name="c", subcore_axis_name="s")
    out = pl.kernel(
        _sc_expand,
        mesh=mesh,
        compiler_params=pltpu.CompilerParams(needs_layout_passes=False),
        out_type=jax.ShapeDtypeStruct((Bn, Cc, N), jnp.float32),
        scratch_types=[
            pltpu.VMEM((24 * 256,), jnp.float32),
            pltpu.VMEM((256,), jnp.int32),
            pltpu.VMEM((24, N), jnp.float32),
            pltpu.SemaphoreType.DMA,
        ],
    )(pooled.reshape(Bn, Cc * 256), idx.reshape(Bn * 256))
    return out.reshape(Bn, Cc, H, W)


# speedup vs baseline: 1.1776x; 1.1776x over previous
import functools

import jax
import jax.numpy as jnp
from jax import lax
from jax.experimental import pallas as pl
from jax.experimental.pallas import tpu as pltpu
from jax.experimental.pallas import tpu_sc as plsc

_NC = 2
_NS = 16
_NW = _NC * _NS


_CPAD = 256


def _stage1_kernel(attn_ref, c_ref, idx_ref, pooled_ref):
    coarse = attn_ref[0, 0]
    for h in range(1, 8):
        coarse = coarse + attn_ref[0, h]
    coarse = coarse * 0.125

    idx = jnp.argmax(coarse, axis=1, keepdims=True)
    idx_ref[0] = idx + pl.program_id(0) * 256

    n = lax.broadcasted_iota(jnp.int32, (4096, 256), 0)
    l = lax.broadcasted_iota(jnp.int32, (4096, 256), 1)
    s = (((n // 256) * 16 + (n % 64) // 4) == l).astype(jnp.float32)
    cc = c_ref[0]
    cc = jnp.concatenate(
        [cc, jnp.zeros((_CPAD - cc.shape[0], cc.shape[1]), cc.dtype)], axis=0)
    pooled_ref[0] = lax.dot_general(
        s, cc, (((0,), (1,)), ((), ())),
        preferred_element_type=jnp.float32)


def _sc_gather(table_hbm, idx_hbm, out_hbm, idx_v, rows_v, sem):
    wid = lax.axis_index("s") * _NC + lax.axis_index("c")
    rows_per_w = 1024 // _NW
    base = wid * rows_per_w
    pltpu.sync_copy(idx_hbm.at[pl.ds(base, rows_per_w)], idx_v)
    pltpu.async_copy(table_hbm.at[idx_v], rows_v, sem).wait()
    pltpu.sync_copy(rows_v, out_hbm.at[pl.ds(base, rows_per_w)])


def _stage3_kernel(g_ref, out_ref):
    n = lax.broadcasted_iota(jnp.int32, (4096, 256), 0)
    l = lax.broadcasted_iota(jnp.int32, (4096, 256), 1)
    s = (((n // 256) * 16 + (n % 64) // 4) == l).astype(jnp.float32)
    out = lax.dot_general(
        g_ref[0], s, (((0,), (1,)), ((), ())),
        preferred_element_type=jnp.float32)
    out_ref[0] = out[:192] * 0.0625


def kernel(A, B, C, D, attn):
    Bn, Cc, H, W = C.shape
    N = H * W
    c2 = C.reshape(Bn, Cc, N)

    idx, pooled = pl.pallas_call(
        _stage1_kernel,
        grid=(Bn,),
        in_specs=[
            pl.BlockSpec((1, 8, 256, 256), lambda b: (b, 0, 0, 0)),
            pl.BlockSpec((1, Cc, N), lambda b: (b, 0, 0)),
        ],
        out_specs=[
            pl.BlockSpec((1, 256, 1), lambda b: (b, 0, 0)),
            pl.BlockSpec((1, 256, _CPAD), lambda b: (b, 0, 0)),
        ],
        out_shape=[
            jax.ShapeDtypeStruct((Bn, 256, 1), jnp.int32),
            jax.ShapeDtypeStruct((Bn, 256, _CPAD), jnp.float32),
        ],
    )(attn, c2)

    table = pooled.reshape(Bn * 256, _CPAD)
    idx_flat = idx.reshape(Bn * 256)

    mesh = plsc.VectorSubcoreMesh(core_axis_name="c", subcore_axis_name="s")
    rows_per_w = (Bn * 256) // _NW
    sc_gather = functools.partial(
        pl.kernel,
        mesh=mesh,
        out_type=jax.ShapeDtypeStruct((Bn * 256, _CPAD), jnp.float32),
        scratch_types=[
            pltpu.VMEM((rows_per_w,), jnp.int32),
            pltpu.VMEM((rows_per_w, _CPAD), jnp.float32),
            pltpu.SemaphoreType.DMA,
        ],
    )(_sc_gather)
    g = sc_gather(table, idx_flat)

    out = pl.pallas_call(
        _stage3_kernel,
        grid=(Bn,),
        in_specs=[pl.BlockSpec((1, 256, _CPAD), lambda b: (b, 0, 0))],
        out_specs=pl.BlockSpec((1, Cc, N), lambda b: (b, 0, 0)),
        out_shape=jax.ShapeDtypeStruct((Bn, Cc, N), jnp.float32),
    )(g.reshape(Bn, 256, _CPAD))
    return out.reshape(Bn, Cc, H, W)
